# Initial kernel scaffold; baseline (speedup 1.0000x reference)
import jax
import jax.numpy as jnp
from jax.experimental import pallas as pl


def kernel(x, edge_index, w, W_edge, b_edge, W_node, b_node, W_c1, b_c1, W_c2, b_c2, aff_W0, aff_b0, syn_W0, syn_b0, aff_W1, aff_b1, syn_W1, syn_b1):
    raise NotImplementedError("write your pallas kernel here")


# speedup vs baseline: 2.8949x; 2.8949x over previous
import functools

import jax
import jax.numpy as jnp
import numpy as np
from jax import lax
from jax.experimental import pallas as pl
from jax.experimental.pallas import tpu as pltpu
from jax.experimental.pallas import tpu_sc as plsc

N = 10000
E = 320000
H = 128
RANK = 10
WDIM = 128

NC = 2
NS = 16
NW = NC * NS
CH = 128
E_PAD = 327680
NCH = E_PAD // (NW * CH)
ACC_ROWS = 10240
PAD_ROW = N


def _sc_segment_sum(src3, dst3, W_edge):
  mesh = plsc.VectorSubcoreMesh(core_axis_name="c", subcore_axis_name="s")

  @functools.partial(
      pl.kernel,
      out_type=jax.ShapeDtypeStruct((NC, ACC_ROWS, H), jnp.float32),
      mesh=mesh,
      scratch_types=[
          pltpu.VMEM((NCH, CH), jnp.int32),
          pltpu.VMEM((NCH, CH), jnp.int32),
          pltpu.VMEM((CH, H), jnp.float32),
          pltpu.VMEM_SHARED((ACC_ROWS, H), jnp.float32),
          pltpu.SemaphoreType.DMA,
      ],
  )
  def seg_kernel(src_hbm, dst_hbm, wedge_hbm, out_hbm, src_v, dst_v, rows_v,
                 acc_sh, sem):
    c = lax.axis_index("c")
    s = lax.axis_index("s")
    wid = s * NC + c

    pltpu.sync_copy(src_hbm.at[wid], src_v)
    pltpu.sync_copy(dst_hbm.at[wid], dst_v)

    def zrow(r, _):
      for cc in range(H // 16):
        rows_v[r, pl.ds(cc * 16, 16)] = jnp.zeros((16,), jnp.float32)
      return 0

    lax.fori_loop(0, CH, zrow, 0)
    for k in range(ACC_ROWS // (NS * CH)):
      pltpu.sync_copy(rows_v, acc_sh.at[pl.ds((s * 5 + k) * CH, CH)])
    plsc.subcore_barrier()

    def body(j, _):
      pltpu.async_copy(wedge_hbm.at[src_v.at[j]], rows_v, sem).wait()
      pltpu.sync_copy(rows_v, acc_sh.at[dst_v.at[j]], add=True)
      return 0

    lax.fori_loop(0, NCH, body, 0)
    plsc.subcore_barrier()

    rows_per_tile = ACC_ROWS // NS
    pltpu.sync_copy(acc_sh.at[pl.ds(s * rows_per_tile, rows_per_tile)],
                    out_hbm.at[c, pl.ds(s * rows_per_tile, rows_per_tile)])

  return seg_kernel(src3, dst3, W_edge)[:, :N, :]


def _dotT(a, b):
  return lax.dot_general(a, b, (((1,), (1,)), ((), ())),
                         preferred_element_type=jnp.float32)


def _leaky(x):
  return jnp.maximum(x, 0.01 * x)


BLK = 1000


def _dense_kernel(p_ref, x_ref, be_ref, wc1_ref, bc1_ref, wn_ref, bn_ref,
                  wc2_ref, bc2_ref, w_ref, alw0_ref, arw0_ref, alb0_ref,
                  arb0_ref, syn0_ref, sb0_ref, alw1_ref, arw1_ref, alb1_ref,
                  arb1_ref, syn1_ref, sb1_ref, out_ref, wm0_s, wm1_s):
  pid = pl.program_id(0)

  @pl.when(pid == 0)
  def _compute_styles():
    for (lyr, alw, arw, alb, arb, syn, wm_s) in (
        (0, alw0_ref, arw0_ref, alb0_ref, arb0_ref, syn0_ref, wm0_s),
        (1, alw1_ref, arw1_ref, alb1_ref, arb1_ref, syn1_ref, wm1_s)):
      wv = w_ref[lyr:lyr + 1, :]
      lrows = [_dotT(wv, alw[r]) + alb[r] for r in range(RANK)]
      rrows = [_dotT(wv, arw[r]) + arb[r] for r in range(RANK)]
      lt = jnp.concatenate(lrows, axis=0)
      rt = jnp.concatenate(rrows, axis=0)
      mod = lax.dot_general(lt, rt, (((0,), (0,)), ((), ())),
                            preferred_element_type=jnp.float32)
      wm = syn[...] * (mod * np.float32(1.0 / np.sqrt(RANK)) + 1.0)
      nrm = jnp.sqrt(jnp.sum(wm * wm, axis=1, keepdims=True)) + 1e-8
      wm_s[...] = wm / nrm

  p = p_ref[0] + p_ref[1] + be_ref[...]
  out = p + _dotT(p, wc1_ref[...]) + bc1_ref[...]
  xh = _dotT(x_ref[...], wn_ref[...]) + bn_ref[...]
  out = out + xh + _dotT(xh, wc2_ref[...]) + bc2_ref[...]
  out = _leaky(out)
  out = _leaky(_dotT(out, wm0_s[...]) + sb0_ref[...])
  out_ref[...] = _leaky(_dotT(out, wm1_s[...]) + sb1_ref[...])


def _dense(p, x, b_edge, W_c1, b_c1, W_node, b_node, W_c2, b_c2, w, alw0,
           arw0, alb0, arb0, syn_W0, syn_b0, alw1, arw1, alb1, arb1, syn_W1,
           syn_b1):
  full2 = lambda shape: pl.BlockSpec(shape, lambda i: (0, 0))
  full3 = lambda shape: pl.BlockSpec(shape, lambda i: (0, 0, 0))
  grid = N // BLK
  return pl.pallas_call(
      _dense_kernel,
      grid=(grid,),
      in_specs=[
          pl.BlockSpec((NC, BLK, H), lambda i: (0, i, 0)),
          pl.BlockSpec((BLK, H), lambda i: (i, 0)),
          full2((1, H)), full2((H, H)), full2((1, H)),
          full2((H, H)), full2((1, H)),
          full2((H, H)), full2((1, H)),
          full2((2, WDIM)),
          full3((RANK, H, WDIM)), full3((RANK, H, WDIM)),
          full3((RANK, 1, H)), full3((RANK, 1, H)),
          full2((H, H)), full2((1, H)),
          full3((RANK, H, WDIM)), full3((RANK, H, WDIM)),
          full3((RANK, 1, H)), full3((RANK, 1, H)),
          full2((H, H)), full2((1, H)),
      ],
      out_specs=pl.BlockSpec((BLK, H), lambda i: (i, 0)),
      out_shape=jax.ShapeDtypeStruct((N, H), jnp.float32),
      scratch_shapes=[
          pltpu.VMEM((H, H), jnp.float32),
          pltpu.VMEM((H, H), jnp.float32),
      ],
  )(p, x, b_edge, W_c1, b_c1, W_node, b_node, W_c2, b_c2, w, alw0, arw0,
    alb0, arb0, syn_W0, syn_b0, alw1, arw1, alb1, arb1, syn_W1, syn_b1)


def kernel(x, edge_index, w, W_edge, b_edge, W_node, b_node, W_c1, b_c1,
           W_c2, b_c2, aff_W0, aff_b0, syn_W0, syn_b0, aff_W1, aff_b1,
           syn_W1, syn_b1):
  src = edge_index[0]
  dst = edge_index[1]
  pad = E_PAD - E
  src3 = jnp.concatenate([src, jnp.zeros((pad,), jnp.int32)]).reshape(
      NW, NCH, CH)
  dst3 = jnp.concatenate([dst, jnp.full((pad,), PAD_ROW, jnp.int32)]).reshape(
      NW, NCH, CH)

  p = _sc_segment_sum(src3, dst3, W_edge)

  def split_aff(aff_W, aff_b):
    alw = aff_W[:H * RANK].reshape(H, RANK, WDIM).transpose(1, 0, 2)
    arw = aff_W[H * RANK:].reshape(RANK, H, WDIM)
    alb = aff_b[:H * RANK].reshape(H, RANK).T.reshape(RANK, 1, H)
    arb = aff_b[H * RANK:].reshape(RANK, 1, H)
    return alw, arw, alb, arb

  alw0, arw0, alb0, arb0 = split_aff(aff_W0, aff_b0)
  alw1, arw1, alb1, arb1 = split_aff(aff_W1, aff_b1)

  return _dense(p, x, b_edge.reshape(1, H), W_c1, b_c1.reshape(1, H),
                W_node, b_node.reshape(1, H), W_c2, b_c2.reshape(1, H), w,
                alw0, arw0, alb0, arb0, syn_W0, syn_b0, alw1, arw1, alb1,
                arb1, syn_W1, syn_b1)


# speedup vs baseline: 3.1891x; 1.1016x over previous
import functools

import jax
import jax.numpy as jnp
import numpy as np
from jax import lax
from jax.experimental import pallas as pl
from jax.experimental.pallas import tpu as pltpu
from jax.experimental.pallas import tpu_sc as plsc

N = 10000
E = 320000
H = 128
RANK = 10
WDIM = 128

NC = 2
NS = 16
NW = NC * NS
CH = 128
NCH = 80
G = 16
NG = NCH // G
E_PAD = NW * NCH * CH
ACC_ROWS = 10240
PAD_ROW = N


def _sc_segment_sum(src3, dst3, W_edge):
  mesh = plsc.VectorSubcoreMesh(core_axis_name="c", subcore_axis_name="s")

  @functools.partial(
      pl.kernel,
      out_type=jax.ShapeDtypeStruct((NC, ACC_ROWS, H), jnp.float32),
      mesh=mesh,
      scratch_types=[
          pltpu.VMEM((2, 2 * G, CH), jnp.int32),
          pltpu.VMEM((CH, H), jnp.float32),
          pltpu.VMEM((CH, H), jnp.float32),
          pltpu.VMEM_SHARED((ACC_ROWS, H), jnp.float32),
          pltpu.SemaphoreType.DMA,
          pltpu.SemaphoreType.DMA,
          pltpu.SemaphoreType.DMA,
      ],
  )
  def seg_kernel(src_hbm, dst_hbm, wedge_hbm, out_hbm, idx_v, rows_a,
                 rows_b, acc_sh, sem_a, sem_b, isem):
    c = lax.axis_index("c")
    s = lax.axis_index("s")
    wid = s * NC + c

    def load_idx_group(g, buf):
      pltpu.async_copy(src_hbm.at[wid, pl.ds(g * G, G)],
                       idx_v.at[buf, pl.ds(0, G)], isem)
      pltpu.async_copy(dst_hbm.at[wid, pl.ds(g * G, G)],
                       idx_v.at[buf, pl.ds(G, G)], isem)

    def wait_idx_group(buf):
      for half in range(2):
        pltpu.make_async_copy(src_hbm.at[wid, pl.ds(0, G)],
                              idx_v.at[buf, pl.ds(half * G, G)], isem).wait()

    def zrow(r, _):
      for cc in range(H // 16):
        rows_a[r, pl.ds(cc * 16, 16)] = jnp.zeros((16,), jnp.float32)
      return 0

    lax.fori_loop(0, CH, zrow, 0)
    zrows = ACC_ROWS // NS
    for k in range(zrows // CH):
      pltpu.sync_copy(rows_a, acc_sh.at[pl.ds(s * zrows + k * CH, CH)])
    plsc.subcore_barrier()

    bufs = ((rows_a, sem_a), (rows_b, sem_b))

    def start_gather(buf, k, b):
      rows, sem = bufs[b]
      pltpu.async_copy(wedge_hbm.at[idx_v.at[buf, k]], rows, sem)

    load_idx_group(0, 0)
    wait_idx_group(0)
    start_gather(0, 0, 0)
    start_gather(0, 1, 1)
    if NG > 1:
      load_idx_group(1, 1)

    def group_body(g, _):
      pb = lax.rem(g, 2)
      qb = lax.rem(g + 1, 2)
      for k in range(G):
        b = k % 2
        rows, sem = bufs[b]
        pltpu.make_async_copy(wedge_hbm.at[idx_v.at[pb, k]], rows, sem).wait()
        pltpu.sync_copy(rows, acc_sh.at[idx_v.at[pb, G + k]], add=True)
        if k <= G - 3:
          start_gather(pb, k + 2, b)
        elif k == G - 2:
          @pl.when(g + 1 < NG)
          def _next0():
            wait_idx_group(qb)
            start_gather(qb, 0, b)
        else:
          @pl.when(g + 1 < NG)
          def _next1():
            start_gather(qb, 1, b)

          @pl.when(g + 2 < NG)
          def _load_ahead():
            load_idx_group(g + 2, pb)
      return 0

    lax.fori_loop(0, NG, group_body, 0)
    plsc.subcore_barrier()

    rows_per_tile = ACC_ROWS // NS
    pltpu.sync_copy(acc_sh.at[pl.ds(s * rows_per_tile, rows_per_tile)],
                    out_hbm.at[c, pl.ds(s * rows_per_tile, rows_per_tile)])

  return seg_kernel(src3, dst3, W_edge)


def _dotT(a, b):
  return lax.dot_general(a, b, (((1,), (1,)), ((), ())),
                         preferred_element_type=jnp.float32)


def _leaky(x):
  return jnp.maximum(x, 0.01 * x)


BLK = 1000


def _dense_kernel(p_ref, x_ref, be_ref, wc1_ref, bc1_ref, wn_ref, bn_ref,
                  wc2_ref, bc2_ref, w_ref, alw0_ref, arw0_ref, alb0_ref,
                  arb0_ref, syn0_ref, sb0_ref, alw1_ref, arw1_ref, alb1_ref,
                  arb1_ref, syn1_ref, sb1_ref, out_ref, wm0_s, wm1_s):
  pid = pl.program_id(0)

  @pl.when(pid == 0)
  def _compute_styles():
    for (lyr, alw, arw, alb, arb, syn, wm_s) in (
        (0, alw0_ref, arw0_ref, alb0_ref, arb0_ref, syn0_ref, wm0_s),
        (1, alw1_ref, arw1_ref, alb1_ref, arb1_ref, syn1_ref, wm1_s)):
      wv = w_ref[lyr:lyr + 1, :]
      lrows = [_dotT(wv, alw[r]) + alb[r] for r in range(RANK)]
      rrows = [_dotT(wv, arw[r]) + arb[r] for r in range(RANK)]
      lt = jnp.concatenate(lrows, axis=0)
      rt = jnp.concatenate(rrows, axis=0)
      mod = lax.dot_general(lt, rt, (((0,), (0,)), ((), ())),
                            preferred_element_type=jnp.float32)
      wm = syn[...] * (mod * np.float32(1.0 / np.sqrt(RANK)) + 1.0)
      nrm = jnp.sqrt(jnp.sum(wm * wm, axis=1, keepdims=True)) + 1e-8
      wm_s[...] = wm / nrm

  p = p_ref[0] + p_ref[1] + be_ref[...]
  out = p + _dotT(p, wc1_ref[...]) + bc1_ref[...]
  xh = _dotT(x_ref[...], wn_ref[...]) + bn_ref[...]
  out = out + xh + _dotT(xh, wc2_ref[...]) + bc2_ref[...]
  out = _leaky(out)
  out = _leaky(_dotT(out, wm0_s[...]) + sb0_ref[...])
  out_ref[...] = _leaky(_dotT(out, wm1_s[...]) + sb1_ref[...])


def _dense(p, x, b_edge, W_c1, b_c1, W_node, b_node, W_c2, b_c2, w, alw0,
           arw0, alb0, arb0, syn_W0, syn_b0, alw1, arw1, alb1, arb1, syn_W1,
           syn_b1):
  full2 = lambda shape: pl.BlockSpec(shape, lambda i: (0, 0))
  full3 = lambda shape: pl.BlockSpec(shape, lambda i: (0, 0, 0))
  grid = N // BLK
  return pl.pallas_call(
      _dense_kernel,
      grid=(grid,),
      in_specs=[
          pl.BlockSpec((NC, BLK, H), lambda i: (0, i, 0)),
          pl.BlockSpec((BLK, H), lambda i: (i, 0)),
          full2((1, H)), full2((H, H)), full2((1, H)),
          full2((H, H)), full2((1, H)),
          full2((H, H)), full2((1, H)),
          full2((2, WDIM)),
          full3((RANK, H, WDIM)), full3((RANK, H, WDIM)),
          full3((RANK, 1, H)), full3((RANK, 1, H)),
          full2((H, H)), full2((1, H)),
          full3((RANK, H, WDIM)), full3((RANK, H, WDIM)),
          full3((RANK, 1, H)), full3((RANK, 1, H)),
          full2((H, H)), full2((1, H)),
      ],
      out_specs=pl.BlockSpec((BLK, H), lambda i: (i, 0)),
      out_shape=jax.ShapeDtypeStruct((N, H), jnp.float32),
      scratch_shapes=[
          pltpu.VMEM((H, H), jnp.float32),
          pltpu.VMEM((H, H), jnp.float32),
      ],
  )(p, x, b_edge, W_c1, b_c1, W_node, b_node, W_c2, b_c2, w, alw0, arw0,
    alb0, arb0, syn_W0, syn_b0, alw1, arw1, alb1, arb1, syn_W1, syn_b1)


def kernel(x, edge_index, w, W_edge, b_edge, W_node, b_node, W_c1, b_c1,
           W_c2, b_c2, aff_W0, aff_b0, syn_W0, syn_b0, aff_W1, aff_b1,
           syn_W1, syn_b1):
  src = edge_index[0]
  dst = edge_index[1]
  pad = E_PAD - E
  src3 = jnp.concatenate([src, jnp.zeros((pad,), jnp.int32)]).reshape(
      NW, NCH, CH)
  dst3 = jnp.concatenate([dst, jnp.full((pad,), PAD_ROW, jnp.int32)]).reshape(
      NW, NCH, CH)

  p = _sc_segment_sum(src3, dst3, W_edge)

  def split_aff(aff_W, aff_b):
    alw = aff_W[:H * RANK].reshape(H, RANK, WDIM).transpose(1, 0, 2)
    arw = aff_W[H * RANK:].reshape(RANK, H, WDIM)
    alb = aff_b[:H * RANK].reshape(H, RANK).T.reshape(RANK, 1, H)
    arb = aff_b[H * RANK:].reshape(RANK, 1, H)
    return alw, arw, alb, arb

  alw0, arw0, alb0, arb0 = split_aff(aff_W0, aff_b0)
  alw1, arw1, alb1, arb1 = split_aff(aff_W1, aff_b1)

  return _dense(p, x, b_edge.reshape(1, H), W_c1, b_c1.reshape(1, H),
                W_node, b_node.reshape(1, H), W_c2, b_c2.reshape(1, H), w,
                alw0, arw0, alb0, arb0, syn_W0, syn_b0, alw1, arw1, alb1,
                arb1, syn_W1, syn_b1)
